# Initial kernel scaffold; baseline (speedup 1.0000x reference)
import jax
import jax.numpy as jnp
from jax.experimental import pallas as pl


def kernel(data, labels, x, var_alpha, steps):
    raise NotImplementedError("write your pallas kernel here")


# speedup vs baseline: 248.5050x; 248.5050x over previous
import functools

import jax
import jax.numpy as jnp
from jax import lax
from jax.experimental import pallas as pl
from jax.experimental.pallas import tpu as pltpu
from jax.experimental.pallas import tpu_sc as plsc

_B = 16
_TW = 25
_NX = 2048
_NT = 250
_NNEI = 2
_TMIN, _TMAX = 0.0, 4.0
_N = _B * _NX
_EB = 2 * _NNEI * _NX - _NNEI * (_NNEI + 1)
_EPAD = 8192
_F = 2 * _TW + 3
_L = 16
_HALO = 16

_DCANDS = (-2, -1, 1, 2)


def _edge_body(x_hbm, out_hbm, xpad_v, src_v, dst_v):
    nc = 2
    wid = lax.axis_index("s") * nc + lax.axis_index("c")

    @pl.when(wid < _B)
    def _():
        b = wid
        sentinel = jnp.full((_L,), -1e9, jnp.float32)
        xpad_v[pl.ds(0, _L)] = sentinel
        xpad_v[pl.ds(_HALO + _NX, _L)] = sentinel
        pltpu.sync_copy(x_hbm.at[pl.ds(b * _NX, _NX)], xpad_v.at[pl.ds(_HALO, _NX)])

        iota = lax.iota(jnp.int32, _L)
        ones = jnp.full((_L,), 1, jnp.int32)
        zeros = jnp.zeros((_L,), jnp.int32)
        fone = jnp.full((_L,), 1.0, jnp.float32)
        fzero = jnp.zeros((_L,), jnp.float32)
        lane0 = jnp.where(iota == 0, fone, fzero)
        x01 = xpad_v[pl.ds(_HALO, _L)]
        x12 = xpad_v[pl.ds(_HALO + 1, _L)]
        dx = jnp.sum((x12 - x01) * lane0)
        radius = _NNEI * dx + dx * 0.1

        def body(it, carry):
            i0 = it * _L
            xi = xpad_v[pl.ds(_HALO + i0, _L)]
            base_i = b * _NX + i0 + iota
            masks = []
            cnt = jnp.zeros((_L,), jnp.int32)
            for d in _DCANDS:
                xj = xpad_v[pl.ds(_HALO + i0 + d, _L)]
                m = jnp.abs(xj - xi) <= radius
                masks.append(m)
                cnt = cnt + jnp.where(m, ones, zeros)
            incl = plsc.cumsum(cnt)
            pos_base = carry + incl - cnt
            off = jnp.zeros((_L,), jnp.int32)
            for d, m in zip(_DCANDS, masks):
                pos = pos_base + off
                plsc.store_scatter(src_v, [pos], base_i + d, mask=m)
                plsc.store_scatter(dst_v, [pos], base_i, mask=m)
                off = off + jnp.where(m, ones, zeros)
            return carry + jnp.sum(cnt)

        lax.fori_loop(0, _NX // _L, body, jnp.int32(0))
        pltpu.sync_copy(src_v, out_hbm.at[pl.ds((2 * b) * _EPAD, _EPAD)])
        pltpu.sync_copy(dst_v, out_hbm.at[pl.ds((2 * b + 1) * _EPAD, _EPAD)])


@jax.jit
def _edge_call(xr):
    mesh = plsc.VectorSubcoreMesh(core_axis_name="c", subcore_axis_name="s")
    fn = functools.partial(
        pl.kernel,
        mesh=mesh,
        out_type=jax.ShapeDtypeStruct((_B * 2 * _EPAD,), jnp.int32),
        scratch_types=[
            pltpu.VMEM((2 * _HALO + _NX,), jnp.float32),
            pltpu.VMEM((_EPAD,), jnp.int32),
            pltpu.VMEM((_EPAD,), jnp.int32),
        ],
        compiler_params=pltpu.CompilerParams(needs_layout_passes=False),
    )(_edge_body)
    return fn(xr)


def _feat_body(steps_ref, alpha_ref, data_ref, labels_ref, x_ref, out_ref):
    b = pl.program_id(0)
    u = jnp.transpose(data_ref[...], (0, 2, 1))
    y = jnp.transpose(labels_ref[...], (0, 2, 1))
    t_val = steps_ref[b].astype(jnp.float32) * ((_TMAX - _TMIN) / (_NT - 1))
    tcol = jnp.full((1, _NX, 1), t_val, jnp.float32)
    xcol = x_ref[...].reshape(1, _NX, 1)
    acol = jnp.full((1, _NX, 1), alpha_ref[b], jnp.float32)
    out_ref[...] = jnp.concatenate([u, y, tcol, xcol, acol], axis=2)


@jax.jit
def _feat_call(steps, var_alpha, data, labels, xr3):
    return pl.pallas_call(
        _feat_body,
        grid=(_B,),
        in_specs=[
            pl.BlockSpec(memory_space=pltpu.SMEM),
            pl.BlockSpec(memory_space=pltpu.SMEM),
            pl.BlockSpec((1, _TW, _NX), lambda b: (b, 0, 0)),
            pl.BlockSpec((1, _TW, _NX), lambda b: (b, 0, 0)),
            pl.BlockSpec((1, 1, _NX), lambda b: (b, 0, 0)),
        ],
        out_specs=pl.BlockSpec((1, _NX, _F), lambda b: (b, 0, 0)),
        out_shape=jax.ShapeDtypeStruct((_B, _NX, _F), jnp.float32),
    )(steps, var_alpha, data, labels, xr3)


def kernel(data, labels, x, var_alpha, steps):
    node_feat = _feat_call(steps, var_alpha, data, labels, x.reshape(_B, 1, _NX))
    ebuf = _edge_call(x.reshape(_B * _NX)).reshape(_B, 2, _EPAD)
    node_feat = node_feat.reshape(_N, _F)
    edge_index = ebuf[:, :, :_EB].transpose(1, 0, 2).reshape(2, _B * _EB)
    return node_feat, edge_index
